# baseline (speedup 1.0000x reference)
import functools

import jax
import jax.numpy as jnp
from jax.experimental import pallas as pl
from jax.experimental.pallas import tpu as pltpu

ALPHA = 0.25
GAMMA = 2.0
W_CE = 2.0
W_BBOX = 5.0
W_GIOU = 2.0

_CHUNK = 4


def _xyxy_cols(bx):
    cx = bx[:, 0:1]
    cy = bx[:, 1:2]
    w = bx[:, 2:3]
    h = bx[:, 3:4]
    return cx - 0.5 * w, cy - 0.5 * h, cx + 0.5 * w, cy + 0.5 * h


def _loss_kernel(x_hbm_ref, boxes_ref, tb_ref, sidx_ref, lab_ref,
                 out_ref, x_scr, copy_sem, *, nb):
    nbatch, q, c = x_scr.shape
    t = tb_ref.shape[1]

    for b in range(nbatch):
        pltpu.make_async_copy(x_hbm_ref.at[b], x_scr.at[b], copy_sem.at[b]).start()

    boxes_bf = boxes_ref[...].astype(jnp.bfloat16)

    sidx_all = sidx_ref[...]
    lab_all = lab_ref[...]
    sidx_tr = jnp.transpose(sidx_all, (1, 0))
    lab_tr = jnp.transpose(lab_all, (1, 0))

    iota_q1 = jax.lax.broadcasted_iota(jnp.int32, (t, q), 1)
    iota_c1 = jax.lax.broadcasted_iota(jnp.int32, (t, c), 1)
    iota_t0 = jax.lax.broadcasted_iota(jnp.int32, (t, t), 0)
    iota_t1 = jax.lax.broadcasted_iota(jnp.int32, (t, t), 1)
    earlier = iota_t1 < iota_t0

    oh_list = []
    keep_cols = []
    l1_sum = 0.0
    giou_sum = 0.0
    for b in range(nbatch):
        sidx_b = sidx_tr[:, b:b + 1]
        lab_b = lab_tr[:, b:b + 1]
        oh_tq = (sidx_b == iota_q1).astype(jnp.bfloat16)
        oh_list.append(oh_tq)

        key_col = sidx_b * c + lab_b
        key_row = sidx_all[b:b + 1, :] * c + lab_all[b:b + 1, :]
        dup = jnp.max(jnp.where((key_col == key_row) & earlier, 1.0, 0.0),
                      axis=1, keepdims=True)
        keep_cols.append(1.0 - dup)

        pb = jax.lax.dot_general(oh_tq, boxes_bf[b], (((1,), (0,)), ((), ())),
                                 preferred_element_type=jnp.float32)
        tb = tb_ref[b]
        l1_sum = l1_sum + jnp.sum(jnp.abs(pb - tb))

        px1, py1, px2, py2 = _xyxy_cols(pb)
        tx1, ty1, tx2, ty2 = _xyxy_cols(tb)
        area_p = (px2 - px1) * (py2 - py1)
        area_t = (tx2 - tx1) * (ty2 - ty1)
        iw = jnp.clip(jnp.minimum(px2, tx2) - jnp.maximum(px1, tx1), 0.0, None)
        ih = jnp.clip(jnp.minimum(py2, ty2) - jnp.maximum(py1, ty1), 0.0, None)
        inter = iw * ih
        union = area_p + area_t - inter
        iou = inter / union
        ew = jnp.clip(jnp.maximum(px2, tx2) - jnp.minimum(px1, tx1), 0.0, None)
        eh = jnp.clip(jnp.maximum(py2, ty2) - jnp.minimum(py1, ty1), 0.0, None)
        earea = ew * eh
        g = iou - (earea - union) / earea
        giou_sum = giou_sum + jnp.sum(1.0 - g)

    v_cols = []
    loss0_parts = []
    start = 0
    for size in (1, 1, 2, 4, 4, 4):
        for b in range(start, start + size):
            pltpu.make_async_copy(x_hbm_ref.at[b], x_scr.at[b],
                                  copy_sem.at[b]).wait()
        xc = x_scr[start:start + size]

        e = jnp.exp(-jnp.abs(xc))
        lse = jnp.log1p(e)
        r = 1.0 / (1.0 + e)
        p = jnp.where(xc >= 0.0, r, 1.0 - r)
        ce0 = jnp.maximum(xc, 0.0) + lse
        loss0_parts.append(jnp.sum((1.0 - ALPHA) * (p * p) * ce0))

        x_bf = xc.astype(jnp.bfloat16)

        for i in range(size):
            b = start + i
            rows = jax.lax.dot_general(oh_list[b], x_bf[i],
                                       (((1,), (0,)), ((), ())),
                                       preferred_element_type=jnp.float32)
            lab_b = lab_tr[:, b:b + 1]
            v_cols.append(jnp.sum(jnp.where(lab_b == iota_c1, rows, 0.0),
                                  axis=1, keepdims=True))
        start += size

    loss0_sum = sum(loss0_parts)

    v = jnp.concatenate(v_cols, axis=1)
    keep = jnp.concatenate(keep_cols, axis=1)
    ev = jnp.exp(-jnp.abs(v))
    lsev = jnp.log1p(ev)
    rv = 1.0 / (1.0 + ev)
    pv = jnp.where(v >= 0.0, rv, 1.0 - rv)
    ce0v = jnp.maximum(v, 0.0) + lsev
    l0v = (1.0 - ALPHA) * (pv * pv) * ce0v
    omv = 1.0 - pv
    delta = ALPHA * (omv * omv) * (ce0v - v) - l0v
    corr_sum = jnp.sum(keep * delta)

    ce_l = (loss0_sum + corr_sum) / nb
    bb_l = l1_sum / nb
    gi_l = giou_sum / nb
    out_ref[0] = ce_l
    out_ref[1] = bb_l
    out_ref[2] = gi_l
    out_ref[3] = W_CE * ce_l + W_BBOX * bb_l + W_GIOU * gi_l


def kernel(pred_logits, pred_boxes, tgt_boxes, tgt_labels, src_idx):
    B, Q, C = pred_logits.shape
    T = tgt_labels.shape[1]
    nb = float(max(1, B * T))

    out = pl.pallas_call(
        functools.partial(_loss_kernel, nb=nb),
        in_specs=[
            pl.BlockSpec(memory_space=pl.ANY),
            pl.BlockSpec(memory_space=pltpu.VMEM),
            pl.BlockSpec(memory_space=pltpu.VMEM),
            pl.BlockSpec(memory_space=pltpu.VMEM),
            pl.BlockSpec(memory_space=pltpu.VMEM),
        ],
        out_specs=pl.BlockSpec(memory_space=pltpu.SMEM),
        out_shape=jax.ShapeDtypeStruct((4,), jnp.float32),
        scratch_shapes=[
            pltpu.VMEM((B, Q, C), jnp.float32),
            pltpu.SemaphoreType.DMA((B,)),
        ],
    )(pred_logits, pred_boxes, tgt_boxes,
      src_idx.astype(jnp.int32), tgt_labels.astype(jnp.int32))

    return (out[0], out[1], out[2], out[3])


# speedup vs baseline: 1.0812x; 1.0812x over previous
import functools

import jax
import jax.numpy as jnp
from jax.experimental import pallas as pl
from jax.experimental.pallas import tpu as pltpu

ALPHA = 0.25
GAMMA = 2.0
W_CE = 2.0
W_BBOX = 5.0
W_GIOU = 2.0

_CHUNK = 4


def _xyxy_cols(bx):
    cx = bx[:, 0:1]
    cy = bx[:, 1:2]
    w = bx[:, 2:3]
    h = bx[:, 3:4]
    return cx - 0.5 * w, cy - 0.5 * h, cx + 0.5 * w, cy + 0.5 * h


def _loss_kernel(x_hbm_ref, boxes_ref, tb_ref, sidx_ref, lab_ref,
                 out_ref, x_scr, copy_sem, *, nb):
    nbatch, q, c = x_scr.shape
    t = tb_ref.shape[1]

    for b in range(nbatch):
        pltpu.make_async_copy(x_hbm_ref.at[b], x_scr.at[b], copy_sem.at[b]).start()

    boxes_bf = boxes_ref[...].astype(jnp.bfloat16)

    sidx_all = sidx_ref[...]
    lab_all = lab_ref[...]
    sidx_tr = jnp.transpose(sidx_all, (1, 0))
    lab_tr = jnp.transpose(lab_all, (1, 0))

    iota_q1 = jax.lax.broadcasted_iota(jnp.int32, (t, q), 1)
    iota_c1 = jax.lax.broadcasted_iota(jnp.int32, (t, c), 1)
    iota_t0 = jax.lax.broadcasted_iota(jnp.int32, (t, t), 0)
    iota_t1 = jax.lax.broadcasted_iota(jnp.int32, (t, t), 1)
    earlier = iota_t1 < iota_t0

    v_cols = []
    keep_cols = []
    loss0_parts = []
    l1_sum = 0.0
    giou_sum = 0.0
    for chunk in range(nbatch // _CHUNK):
        for i in range(_CHUNK):
            b = chunk * _CHUNK + i
            pltpu.make_async_copy(x_hbm_ref.at[b], x_scr.at[b],
                                  copy_sem.at[b]).wait()
        xc = x_scr[chunk * _CHUNK:(chunk + 1) * _CHUNK]

        e = jnp.exp(-jnp.abs(xc))
        lse = jnp.log1p(e)
        r = 1.0 / (1.0 + e)
        p = jnp.where(xc >= 0.0, r, 1.0 - r)
        ce0 = jnp.maximum(xc, 0.0) + lse
        loss0_parts.append(jnp.sum((1.0 - ALPHA) * (p * p) * ce0))

        x_bf = xc.astype(jnp.bfloat16)

        for i in range(_CHUNK):
            b = chunk * _CHUNK + i
            sidx_b = sidx_tr[:, b:b + 1]
            lab_b = lab_tr[:, b:b + 1]
            oh_tq = (sidx_b == iota_q1).astype(jnp.bfloat16)

            rows = jax.lax.dot_general(oh_tq, x_bf[i], (((1,), (0,)), ((), ())),
                                       preferred_element_type=jnp.float32)
            v_cols.append(jnp.sum(jnp.where(lab_b == iota_c1, rows, 0.0),
                                  axis=1, keepdims=True))

            key_col = sidx_b * c + lab_b
            key_row = sidx_all[b:b + 1, :] * c + lab_all[b:b + 1, :]
            dup = jnp.max(jnp.where((key_col == key_row) & earlier, 1.0, 0.0),
                          axis=1, keepdims=True)
            keep_cols.append(1.0 - dup)

            pb = jax.lax.dot_general(oh_tq, boxes_bf[b], (((1,), (0,)), ((), ())),
                                     preferred_element_type=jnp.float32)
            tb = tb_ref[b]
            l1_sum = l1_sum + jnp.sum(jnp.abs(pb - tb))

            px1, py1, px2, py2 = _xyxy_cols(pb)
            tx1, ty1, tx2, ty2 = _xyxy_cols(tb)
            area_p = (px2 - px1) * (py2 - py1)
            area_t = (tx2 - tx1) * (ty2 - ty1)
            iw = jnp.clip(jnp.minimum(px2, tx2) - jnp.maximum(px1, tx1), 0.0, None)
            ih = jnp.clip(jnp.minimum(py2, ty2) - jnp.maximum(py1, ty1), 0.0, None)
            inter = iw * ih
            union = area_p + area_t - inter
            iou = inter / union
            ew = jnp.clip(jnp.maximum(px2, tx2) - jnp.minimum(px1, tx1), 0.0, None)
            eh = jnp.clip(jnp.maximum(py2, ty2) - jnp.minimum(py1, ty1), 0.0, None)
            earea = ew * eh
            g = iou - (earea - union) / earea
            giou_sum = giou_sum + jnp.sum(1.0 - g)

    loss0_sum = sum(loss0_parts)

    v = jnp.concatenate(v_cols, axis=1)
    keep = jnp.concatenate(keep_cols, axis=1)
    ev = jnp.exp(-jnp.abs(v))
    lsev = jnp.log1p(ev)
    rv = 1.0 / (1.0 + ev)
    pv = jnp.where(v >= 0.0, rv, 1.0 - rv)
    ce0v = jnp.maximum(v, 0.0) + lsev
    l0v = (1.0 - ALPHA) * (pv * pv) * ce0v
    omv = 1.0 - pv
    delta = ALPHA * (omv * omv) * (ce0v - v) - l0v
    corr_sum = jnp.sum(keep * delta)

    ce_l = (loss0_sum + corr_sum) / nb
    bb_l = l1_sum / nb
    gi_l = giou_sum / nb
    out_ref[0] = ce_l
    out_ref[1] = bb_l
    out_ref[2] = gi_l
    out_ref[3] = W_CE * ce_l + W_BBOX * bb_l + W_GIOU * gi_l


def kernel(pred_logits, pred_boxes, tgt_boxes, tgt_labels, src_idx):
    B, Q, C = pred_logits.shape
    T = tgt_labels.shape[1]
    nb = float(max(1, B * T))

    out = pl.pallas_call(
        functools.partial(_loss_kernel, nb=nb),
        in_specs=[
            pl.BlockSpec(memory_space=pl.ANY),
            pl.BlockSpec(memory_space=pltpu.VMEM),
            pl.BlockSpec(memory_space=pltpu.VMEM),
            pl.BlockSpec(memory_space=pltpu.VMEM),
            pl.BlockSpec(memory_space=pltpu.VMEM),
        ],
        out_specs=pl.BlockSpec(memory_space=pltpu.SMEM),
        out_shape=jax.ShapeDtypeStruct((4,), jnp.float32),
        scratch_shapes=[
            pltpu.VMEM((B, Q, C), jnp.float32),
            pltpu.SemaphoreType.DMA((B,)),
        ],
    )(pred_logits, pred_boxes, tgt_boxes,
      src_idx.astype(jnp.int32), tgt_labels.astype(jnp.int32))

    return (out[0], out[1], out[2], out[3])


# speedup vs baseline: 1.6354x; 1.5127x over previous
import functools

import jax
import jax.numpy as jnp
from jax.experimental import pallas as pl
from jax.experimental.pallas import tpu as pltpu

ALPHA = 0.25
GAMMA = 2.0
W_CE = 2.0
W_BBOX = 5.0
W_GIOU = 2.0

_CHUNK = 4


def _xyxy_rows(bx):
    cx = bx[0:1, :]
    cy = bx[1:2, :]
    w = bx[2:3, :]
    h = bx[3:4, :]
    return cx - 0.5 * w, cy - 0.5 * h, cx + 0.5 * w, cy + 0.5 * h


def _loss_kernel(x_hbm_ref, boxes_ref, tb_ref, sidx_ref, lab_ref,
                 out_ref, x_scr, copy_sem, *, nb):
    nbatch, q, c = x_scr.shape
    t = tb_ref.shape[2]
    nchunks = nbatch // _CHUNK

    for k in range(nchunks):
        pltpu.make_async_copy(x_hbm_ref.at[pl.ds(k * _CHUNK, _CHUNK)],
                              x_scr.at[pl.ds(k * _CHUNK, _CHUNK)],
                              copy_sem.at[k]).start()

    boxes_bf = boxes_ref[...].astype(jnp.bfloat16)

    sidx_all = sidx_ref[...]
    lab_all = lab_ref[...]
    sidx_tr = jnp.transpose(sidx_all, (1, 0))
    lab_tr = jnp.transpose(lab_all, (1, 0))

    iota_q1 = jax.lax.broadcasted_iota(jnp.int32, (t, q), 1)
    iota_q0 = jax.lax.broadcasted_iota(jnp.int32, (q, t), 0)
    iota_c1 = jax.lax.broadcasted_iota(jnp.int32, (t, c), 1)
    iota_t0 = jax.lax.broadcasted_iota(jnp.int32, (t, t), 0)
    iota_t1 = jax.lax.broadcasted_iota(jnp.int32, (t, t), 1)
    earlier = iota_t1 < iota_t0

    v_cols = []
    keep_cols = []
    loss0_parts = []
    l1_sum = 0.0
    giou_sum = 0.0
    for chunk in range(nchunks):
        pltpu.make_async_copy(x_hbm_ref.at[pl.ds(chunk * _CHUNK, _CHUNK)],
                              x_scr.at[pl.ds(chunk * _CHUNK, _CHUNK)],
                              copy_sem.at[chunk]).wait()
        xc = x_scr[chunk * _CHUNK:(chunk + 1) * _CHUNK]

        e = jnp.exp(-jnp.abs(xc))
        lse = jnp.log1p(e)
        r = 1.0 / (1.0 + e)
        p = jnp.where(xc >= 0.0, r, 1.0 - r)
        ce0 = jnp.maximum(xc, 0.0) + lse
        loss0_parts.append(jnp.sum((1.0 - ALPHA) * (p * p) * ce0))

        x_bf = xc.astype(jnp.bfloat16)

        for i in range(_CHUNK):
            b = chunk * _CHUNK + i
            sidx_b = sidx_tr[:, b:b + 1]
            lab_b = lab_tr[:, b:b + 1]
            oh_tq = (sidx_b == iota_q1).astype(jnp.bfloat16)

            rows = jax.lax.dot_general(oh_tq, x_bf[i], (((1,), (0,)), ((), ())),
                                       preferred_element_type=jnp.float32)
            v_cols.append(jnp.sum(jnp.where(lab_b == iota_c1, rows, 0.0),
                                  axis=1, keepdims=True))

            key_col = sidx_b * c + lab_b
            key_row = sidx_all[b:b + 1, :] * c + lab_all[b:b + 1, :]
            dup = jnp.max(jnp.where((key_col == key_row) & earlier, 1.0, 0.0),
                          axis=1, keepdims=True)
            keep_cols.append(1.0 - dup)

            oh_qt = (jnp.broadcast_to(sidx_all[b:b + 1, :], (q, t)) ==
                     iota_q0).astype(jnp.bfloat16)
            pb = jax.lax.dot_general(boxes_bf[b], oh_qt, (((1,), (0,)), ((), ())),
                                     preferred_element_type=jnp.float32)
            tb = tb_ref[b]
            l1_sum = l1_sum + jnp.sum(jnp.abs(pb - tb))

            px1, py1, px2, py2 = _xyxy_rows(pb)
            tx1, ty1, tx2, ty2 = _xyxy_rows(tb)
            area_p = (px2 - px1) * (py2 - py1)
            area_t = (tx2 - tx1) * (ty2 - ty1)
            iw = jnp.clip(jnp.minimum(px2, tx2) - jnp.maximum(px1, tx1), 0.0, None)
            ih = jnp.clip(jnp.minimum(py2, ty2) - jnp.maximum(py1, ty1), 0.0, None)
            inter = iw * ih
            union = area_p + area_t - inter
            iou = inter / union
            ew = jnp.clip(jnp.maximum(px2, tx2) - jnp.minimum(px1, tx1), 0.0, None)
            eh = jnp.clip(jnp.maximum(py2, ty2) - jnp.minimum(py1, ty1), 0.0, None)
            earea = ew * eh
            g = iou - (earea - union) / earea
            giou_sum = giou_sum + jnp.sum(1.0 - g)

    loss0_sum = sum(loss0_parts)

    v = jnp.concatenate(v_cols, axis=1)
    keep = jnp.concatenate(keep_cols, axis=1)
    ev = jnp.exp(-jnp.abs(v))
    lsev = jnp.log1p(ev)
    rv = 1.0 / (1.0 + ev)
    pv = jnp.where(v >= 0.0, rv, 1.0 - rv)
    ce0v = jnp.maximum(v, 0.0) + lsev
    l0v = (1.0 - ALPHA) * (pv * pv) * ce0v
    omv = 1.0 - pv
    delta = ALPHA * (omv * omv) * (ce0v - v) - l0v
    corr_sum = jnp.sum(keep * delta)

    ce_l = (loss0_sum + corr_sum) / nb
    bb_l = l1_sum / nb
    gi_l = giou_sum / nb
    out_ref[0] = ce_l
    out_ref[1] = bb_l
    out_ref[2] = gi_l
    out_ref[3] = W_CE * ce_l + W_BBOX * bb_l + W_GIOU * gi_l


def kernel(pred_logits, pred_boxes, tgt_boxes, tgt_labels, src_idx):
    B, Q, C = pred_logits.shape
    T = tgt_labels.shape[1]
    nb = float(max(1, B * T))

    out = pl.pallas_call(
        functools.partial(_loss_kernel, nb=nb),
        in_specs=[
            pl.BlockSpec(memory_space=pl.ANY),
            pl.BlockSpec(memory_space=pltpu.VMEM),
            pl.BlockSpec(memory_space=pltpu.VMEM),
            pl.BlockSpec(memory_space=pltpu.VMEM),
            pl.BlockSpec(memory_space=pltpu.VMEM),
        ],
        out_specs=pl.BlockSpec(memory_space=pltpu.SMEM),
        out_shape=jax.ShapeDtypeStruct((4,), jnp.float32),
        scratch_shapes=[
            pltpu.VMEM((B, Q, C), jnp.float32),
            pltpu.SemaphoreType.DMA((B // _CHUNK,)),
        ],
    )(pred_logits, pred_boxes.transpose(0, 2, 1), tgt_boxes.transpose(0, 2, 1),
      src_idx.astype(jnp.int32), tgt_labels.astype(jnp.int32))

    return (out[0], out[1], out[2], out[3])
